# baseline (speedup 1.0000x reference)
import functools
import math

import jax
import jax.numpy as jnp
from jax import lax
from jax.experimental import pallas as pl
from jax.experimental.pallas import tpu as pltpu
from jax.experimental.pallas import tpu_sc as plsc

_NC = 2
_NS = 16
_NW = _NC * _NS

_IDXW = 128


@functools.partial(jax.jit, static_argnums=(2, 3))
def _emb_lookup(table, idx2d, rows_per_chunk, scale):
    n_rows = idx2d.shape[0]
    d = table.shape[1]
    b_total = n_rows * _IDXW
    rows_per_w = n_rows // _NW
    n_chunks = rows_per_w // rows_per_chunk
    ch = rows_per_chunk * _IDXW

    mesh = plsc.VectorSubcoreMesh(core_axis_name="c", subcore_axis_name="s")

    @functools.partial(
        pl.kernel,
        mesh=mesh,
        out_type=jax.ShapeDtypeStruct((b_total, d), jnp.float32),
        scratch_types=[
            pltpu.VMEM((rows_per_chunk, _IDXW), jnp.int32),
            pltpu.VMEM((ch, d), jnp.float32),
            pltpu.SemaphoreType.DMA,
        ],
        compiler_params=pltpu.CompilerParams(use_tc_tiling_on_sc=False),
    )
    def k(table_hbm, idx_hbm, out_hbm, idx_v, rows_v, sem):
        wid = lax.axis_index("s") * _NC + lax.axis_index("c")
        row_base = wid * rows_per_w

        def chunk_body(g, carry):
            row0 = row_base + g * rows_per_chunk
            pltpu.sync_copy(idx_hbm.at[pl.ds(row0, rows_per_chunk)], idx_v)
            copies = []
            for j in range(rows_per_chunk):
                copies.append(
                    pltpu.async_copy(
                        table_hbm.at[idx_v.at[j]],
                        rows_v.at[pl.ds(j * _IDXW, _IDXW)],
                        sem,
                    )
                )
            for c in copies:
                c.wait()

            def mul_body(i, c2):
                for l in range(d // 16):
                    sl = rows_v[i, pl.ds(l * 16, 16)]
                    rows_v[i, pl.ds(l * 16, 16)] = sl * scale
                return c2

            lax.fori_loop(0, ch, mul_body, 0)

            out0 = row0 * _IDXW
            pltpu.sync_copy(rows_v, out_hbm.at[pl.ds(out0, ch)])
            return carry

        lax.fori_loop(0, n_chunks, chunk_body, 0)

    return k(table, idx2d)


def kernel(x, table):
    b = x.shape[0] * x.shape[1]
    d = table.shape[1]
    idx2d = x.reshape(b // _IDXW, _IDXW).astype(jnp.int32)
    scale = float(math.sqrt(d))
    out = _emb_lookup(table, idx2d, 4, scale)
    return out.reshape(x.shape[0], x.shape[1], d)


# speedup vs baseline: 1.1051x; 1.1051x over previous
import functools
import math

import jax
import jax.numpy as jnp
from jax import lax
from jax.experimental import pallas as pl
from jax.experimental.pallas import tpu as pltpu
from jax.experimental.pallas import tpu_sc as plsc

_NC = 2
_NS = 16
_NW = _NC * _NS

_IDXW = 128


@functools.partial(jax.jit, static_argnums=(2, 3))
def _emb_lookup(table, idx2d, rows_per_chunk, scale):
    n_rows = idx2d.shape[0]
    d = table.shape[1]
    b_total = n_rows * _IDXW
    rows_per_w = n_rows // _NW
    n_chunks = rows_per_w // rows_per_chunk
    n_pairs = n_chunks // 2
    ch = rows_per_chunk * _IDXW

    mesh = plsc.VectorSubcoreMesh(core_axis_name="c", subcore_axis_name="s")

    @functools.partial(
        pl.kernel,
        mesh=mesh,
        out_type=jax.ShapeDtypeStruct((b_total, d), jnp.float32),
        scratch_types=[
            pltpu.VMEM((rows_per_w, _IDXW), jnp.int32),
            pltpu.VMEM((2, ch, d), jnp.float32),
            pltpu.SemaphoreType.DMA,
            pltpu.SemaphoreType.DMA,
        ],
        compiler_params=pltpu.CompilerParams(use_tc_tiling_on_sc=False),
    )
    def k(table_hbm, idx_hbm, out_hbm, idx_v, rows_v, sem0, sem1):
        wid = lax.axis_index("s") * _NC + lax.axis_index("c")
        row_base = wid * rows_per_w
        sems = (sem0, sem1)

        pltpu.sync_copy(idx_hbm.at[pl.ds(row_base, rows_per_w)], idx_v)

        def fire(g, b):
            for j in range(rows_per_chunk):
                pltpu.async_copy(
                    table_hbm.at[idx_v.at[g * rows_per_chunk + j]],
                    rows_v.at[b].at[pl.ds(j * _IDXW, _IDXW)],
                    sems[b],
                )

        def drain(b):
            for j in range(rows_per_chunk):
                pltpu.make_async_copy(
                    table_hbm.at[idx_v.at[j]],
                    rows_v.at[b].at[pl.ds(j * _IDXW, _IDXW)],
                    sems[b],
                ).wait()

        def scale_rows(b):
            def mul_body(i, c2):
                for l in range(d // 16):
                    sl = rows_v[b, i, pl.ds(l * 16, 16)]
                    rows_v[b, i, pl.ds(l * 16, 16)] = sl * scale
                return c2

            lax.fori_loop(0, ch, mul_body, 0)

        def store(g, b):
            out0 = (row_base + g * rows_per_chunk) * _IDXW
            pltpu.sync_copy(rows_v.at[b], out_hbm.at[pl.ds(out0, ch)])

        fire(0, 0)

        def pair_body(gp, carry):
            g0 = 2 * gp
            fire(g0 + 1, 1)
            drain(0)
            scale_rows(0)
            store(g0, 0)

            @pl.when(gp + 1 < n_pairs)
            def _():
                fire(g0 + 2, 0)

            drain(1)
            scale_rows(1)
            store(g0 + 1, 1)
            return carry

        lax.fori_loop(0, n_pairs, pair_body, 0)

    return k(table, idx2d)


def kernel(x, table):
    b = x.shape[0] * x.shape[1]
    d = table.shape[1]
    idx2d = x.reshape(b // _IDXW, _IDXW).astype(jnp.int32)
    scale = float(math.sqrt(d))
    out = _emb_lookup(table, idx2d, 5, scale)
    return out.reshape(x.shape[0], x.shape[1], d)
